# Initial kernel scaffold; baseline (speedup 1.0000x reference)
import jax
import jax.numpy as jnp
from jax.experimental import pallas as pl


def kernel(t, beta, alpha):
    raise NotImplementedError("write your pallas kernel here")


# speedup vs baseline: 8.2917x; 8.2917x over previous
import functools

import jax
import jax.numpy as jnp
from jax import lax
from jax.experimental import pallas as pl
from jax.experimental.pallas import tpu as pltpu
from jax.experimental.pallas import tpu_sc as plsc

NUM_T = 1000
BATCH = 16384
L = 16
NC = 2
NS = 16
NW = NC * NS
B_PER_W = BATCH // NW
CHUNKS = B_PER_W // L


def _ddpm_lookup(t, beta, alpha):
    mesh = plsc.VectorSubcoreMesh(core_axis_name="c", subcore_axis_name="s")

    @functools.partial(
        pl.kernel,
        mesh=mesh,
        out_type=(
            jax.ShapeDtypeStruct((BATCH,), jnp.float32),
            jax.ShapeDtypeStruct((BATCH,), jnp.float32),
        ),
        scratch_types=[
            pltpu.VMEM((NUM_T,), jnp.float32),
            pltpu.VMEM((NUM_T,), jnp.float32),
            pltpu.VMEM((B_PER_W,), jnp.int32),
            pltpu.VMEM((B_PER_W,), jnp.float32),
            pltpu.VMEM((B_PER_W,), jnp.float32),
        ],
        compiler_params=pltpu.CompilerParams(needs_layout_passes=False),
    )
    def k(t_hbm, beta_hbm, alpha_hbm, beta_out, alpha_out,
          beta_v, alpha_v, idx_v, ob_v, oa_v):
        wid = lax.axis_index("s") * NC + lax.axis_index("c")
        base = wid * B_PER_W
        pltpu.sync_copy(beta_hbm, beta_v)
        pltpu.sync_copy(alpha_hbm, alpha_v)
        pltpu.sync_copy(t_hbm.at[pl.ds(base, B_PER_W)], idx_v)
        for i in range(CHUNKS):
            idx = idx_v[pl.ds(i * L, L)]
            ob_v[pl.ds(i * L, L)] = plsc.load_gather(beta_v, [idx])
            oa_v[pl.ds(i * L, L)] = plsc.load_gather(alpha_v, [idx])
        pltpu.sync_copy(ob_v, beta_out.at[pl.ds(base, B_PER_W)])
        pltpu.sync_copy(oa_v, alpha_out.at[pl.ds(base, B_PER_W)])

    return k(t, beta, alpha)


def kernel(t, beta, alpha):
    beta_t, alpha_t = _ddpm_lookup(t, beta, alpha)
    return (beta_t, alpha_t)


# speedup vs baseline: 8.7984x; 1.0611x over previous
import functools

import jax
import jax.numpy as jnp
from jax import lax
from jax.experimental import pallas as pl
from jax.experimental.pallas import tpu as pltpu
from jax.experimental.pallas import tpu_sc as plsc

NUM_T = 1000
BATCH = 16384
L = 16
NC = 2
NS = 16
NW = NC * NS
B_PER_W = BATCH // NW
CHUNKS = B_PER_W // L


def _ddpm_lookup(t, beta, alpha):
    mesh = plsc.VectorSubcoreMesh(core_axis_name="c", subcore_axis_name="s")

    @functools.partial(
        pl.kernel,
        mesh=mesh,
        out_type=(
            jax.ShapeDtypeStruct((BATCH,), jnp.float32),
            jax.ShapeDtypeStruct((BATCH,), jnp.float32),
        ),
        scratch_types=[
            pltpu.VMEM((NUM_T,), jnp.float32),
            pltpu.VMEM((NUM_T,), jnp.float32),
            pltpu.VMEM((B_PER_W,), jnp.int32),
            pltpu.VMEM((B_PER_W,), jnp.float32),
            pltpu.VMEM((B_PER_W,), jnp.float32),
            pltpu.SemaphoreType.DMA,
            pltpu.SemaphoreType.DMA,
        ],
        compiler_params=pltpu.CompilerParams(needs_layout_passes=False),
    )
    def k(t_hbm, beta_hbm, alpha_hbm, beta_out, alpha_out,
          beta_v, alpha_v, idx_v, ob_v, oa_v, in_sem, out_sem):
        wid = lax.axis_index("s") * NC + lax.axis_index("c")
        base = wid * B_PER_W
        c1 = pltpu.async_copy(beta_hbm, beta_v, in_sem)
        c2 = pltpu.async_copy(alpha_hbm, alpha_v, in_sem)
        c3 = pltpu.async_copy(t_hbm.at[pl.ds(base, B_PER_W)], idx_v, in_sem)
        c1.wait()
        c2.wait()
        c3.wait()
        for i in range(CHUNKS):
            idx = idx_v[pl.ds(i * L, L)]
            ob_v[pl.ds(i * L, L)] = plsc.load_gather(beta_v, [idx])
            oa_v[pl.ds(i * L, L)] = plsc.load_gather(alpha_v, [idx])
        o1 = pltpu.async_copy(ob_v, beta_out.at[pl.ds(base, B_PER_W)], out_sem)
        o2 = pltpu.async_copy(oa_v, alpha_out.at[pl.ds(base, B_PER_W)], out_sem)
        o1.wait()
        o2.wait()

    return k(t, beta, alpha)


def kernel(t, beta, alpha):
    beta_t, alpha_t = _ddpm_lookup(t, beta, alpha)
    return (beta_t, alpha_t)


# speedup vs baseline: 8.8142x; 1.0018x over previous
import functools

import jax
import jax.numpy as jnp
from jax import lax
from jax.experimental import pallas as pl
from jax.experimental.pallas import tpu as pltpu
from jax.experimental.pallas import tpu_sc as plsc

NUM_T = 1000
BATCH = 16384
L = 16
NC = 2
NS = 16
NW = NC * NS
B_PER_W = BATCH // NW
CHUNKS = B_PER_W // L


def _ddpm_lookup(t, beta, alpha):
    mesh = plsc.VectorSubcoreMesh(core_axis_name="c", subcore_axis_name="s")

    @functools.partial(
        pl.kernel,
        mesh=mesh,
        out_type=(
            jax.ShapeDtypeStruct((BATCH,), jnp.float32),
            jax.ShapeDtypeStruct((BATCH,), jnp.float32),
        ),
        scratch_types=[
            pltpu.VMEM((NUM_T,), jnp.float32),
            pltpu.VMEM((NUM_T,), jnp.float32),
            pltpu.VMEM((B_PER_W,), jnp.int32),
            pltpu.VMEM((B_PER_W,), jnp.float32),
            pltpu.VMEM((B_PER_W,), jnp.float32),
            pltpu.SemaphoreType.DMA,
            pltpu.SemaphoreType.DMA,
        ],
        compiler_params=pltpu.CompilerParams(needs_layout_passes=False),
    )
    def k(t_hbm, beta_hbm, alpha_hbm, beta_out, alpha_out,
          beta_v, alpha_v, idx_v, ob_v, oa_v, in_sem, out_sem):
        wid = lax.axis_index("s") * NC + lax.axis_index("c")
        base = wid * B_PER_W
        c1 = pltpu.async_copy(beta_hbm, beta_v, in_sem)
        c2 = pltpu.async_copy(alpha_hbm, alpha_v, in_sem)
        c3 = pltpu.async_copy(t_hbm.at[pl.ds(base, B_PER_W)], idx_v, in_sem)
        c1.wait()
        c2.wait()
        c3.wait()
        def body(i, carry):
            off = pl.multiple_of(i * L, L)
            idx = idx_v[pl.ds(off, L)]
            ob_v[pl.ds(off, L)] = plsc.load_gather(beta_v, [idx])
            oa_v[pl.ds(off, L)] = plsc.load_gather(alpha_v, [idx])
            return carry

        lax.fori_loop(0, CHUNKS, body, 0, unroll=4)
        o1 = pltpu.async_copy(ob_v, beta_out.at[pl.ds(base, B_PER_W)], out_sem)
        o2 = pltpu.async_copy(oa_v, alpha_out.at[pl.ds(base, B_PER_W)], out_sem)
        o1.wait()
        o2.wait()

    return k(t, beta, alpha)


def kernel(t, beta, alpha):
    beta_t, alpha_t = _ddpm_lookup(t, beta, alpha)
    return (beta_t, alpha_t)
